# Initial kernel scaffold; baseline (speedup 1.0000x reference)
import jax
import jax.numpy as jnp
from jax.experimental import pallas as pl


def kernel(x, y):
    raise NotImplementedError("write your pallas kernel here")


# speedup vs baseline: 5.9569x; 5.9569x over previous
import jax
import jax.numpy as jnp
from jax.experimental import pallas as pl
from jax.experimental.pallas import tpu as pltpu

_N = 8192
_K = 8
_R = 256


def _knn_tile(xq_ref, xt_ref, y_ref, out_ref, d_ref):
    t = pl.program_id(0)
    xq = xq_ref[...]
    xt = xt_ref[...]
    y = y_ref[...]

    g = jax.lax.dot_general(
        xq, xt, (((1,), (0,)), ((), ())),
        preferred_element_type=jnp.float32)
    sq_q = xq[:, 0:1] * xq[:, 0:1] + xq[:, 1:2] * xq[:, 1:2]
    sq_a = xt[0:1, :] * xt[0:1, :] + xt[1:2, :] * xt[1:2, :]
    d = (sq_q + sq_a) - 2.0 * g
    d = jnp.maximum(d, 0.0)

    rows = t * _R + jax.lax.broadcasted_iota(jnp.int32, (_R, 1), 0)
    cols = jax.lax.broadcasted_iota(jnp.int32, (_R, _N), 1)
    d = jnp.where(cols == rows, jnp.inf, d)
    d_ref[...] = d

    s = jnp.zeros((_R, 1), jnp.float32)
    for _ in range(_K):
        d = d_ref[...]
        m = jnp.min(d, axis=1, keepdims=True)
        eq = d == m
        jm = jnp.min(jnp.where(eq, cols, _N), axis=1, keepdims=True)
        onehot = cols == jm
        lab = jnp.sum(jnp.where(onehot, y, 0.0), axis=1, keepdims=True)
        s = s + jnp.where(m < 999.0, lab, 0.0)
        d_ref[...] = jnp.where(onehot, jnp.inf, d)

    out_ref[...] = (s > (_K / 2.0)).astype(jnp.float32)


def kernel(x, y):
    n = x.shape[0]
    yf = y.astype(jnp.float32).reshape(1, n)
    xt = x.T

    out = pl.pallas_call(
        _knn_tile,
        grid=(n // _R,),
        in_specs=[
            pl.BlockSpec((_R, 2), lambda i: (i, 0)),
            pl.BlockSpec((2, n), lambda i: (0, 0)),
            pl.BlockSpec((1, n), lambda i: (0, 0)),
        ],
        out_specs=pl.BlockSpec((_R, 1), lambda i: (i, 0)),
        out_shape=jax.ShapeDtypeStruct((n, 1), jnp.float32),
        scratch_shapes=[pltpu.VMEM((_R, _N), jnp.float32)],
    )(x, xt, yf)
    return out.reshape(n)
